# baseline (speedup 1.0000x reference)
import functools

import jax
import jax.numpy as jnp
from jax import lax
from jax.experimental import pallas as pl
from jax.experimental.pallas import tpu as pltpu
from jax.experimental.pallas import tpu_sc as plsc

_TEMP = 10.0
_NC = 2
_NS = 16
_G = 4


def _make_sc_kernel(R, M):
    Mc = M - 2
    NW = _NC * _NS
    rows_per_w = R // NW
    chunks = rows_per_w // _G
    nvec = M // 16

    mesh = plsc.VectorSubcoreMesh(core_axis_name="c", subcore_axis_name="s")

    @functools.partial(
        pl.kernel,
        out_type=jax.ShapeDtypeStruct((R * M,), jnp.float32),
        mesh=mesh,
        scratch_types=[
            pltpu.VMEM((_G * M,), jnp.float32),
            pltpu.VMEM((_G * M,), jnp.float32),
            pltpu.VMEM((16 + _G * Mc,), jnp.int32),
            pltpu.VMEM((_G * M,), jnp.float32),
            pltpu.VMEM((M,), jnp.float32),
            pltpu.VMEM((M,), jnp.float32),
        ],
    )
    def sc_kernel(x_hbm, o_hbm, t0_hbm, t1_hbm, ci_hbm, out_hbm,
                  x_v, o_v, ci_v, out_v, t0_v, t1_v):
        wid = lax.axis_index("s") * _NC + lax.axis_index("c")
        pltpu.sync_copy(t0_hbm, t0_v)
        pltpu.sync_copy(t1_hbm, t1_v)

        @plsc.parallel_loop(0, M, 16, unroll=8)
        def _scale(i):
            t0_v[pl.ds(i, 16)] = t0_v[pl.ds(i, 16)] * _TEMP
            t1_v[pl.ds(i, 16)] = t1_v[pl.ds(i, 16)] * _TEMP

        iota = lax.iota(jnp.int32, 16)
        ci_v[pl.ds(0, 16)] = jnp.zeros((16,), jnp.int32)

        def emit(off, moff, c16):
            xv = x_v[pl.ds(off, 16)]
            ov = o_v[pl.ds(off, 16)]
            t0 = t0_v[pl.ds(moff, 16)]
            t1 = t1_v[pl.ds(moff, 16)]
            cz = c16 == 0
            v = jnp.where(cz, xv, ov)
            th = jnp.where(cz, t0, t1)
            denom = 1.0 + jnp.exp(th - _TEMP * jnp.abs(v))
            out_v[pl.ds(off, 16)] = v / denom

        def chunk_body(g, carry):
            row0 = wid * rows_per_w + g * _G
            xoff = row0 * M
            cioff = row0 * Mc
            pltpu.sync_copy(x_hbm.at[pl.ds(xoff, _G * M)], x_v)
            pltpu.sync_copy(o_hbm.at[pl.ds(xoff, _G * M)], o_v)
            pltpu.sync_copy(ci_hbm.at[pl.ds(cioff, _G * Mc)],
                            ci_v.at[pl.ds(16, _G * Mc)])
            for j in range(_G):
                cbase = 14 + j * Mc
                c0 = ci_v[pl.ds(cbase, 16)]
                c0 = jnp.where(iota >= 2, c0, 0)
                emit(j * M, 0, c0)

                @plsc.parallel_loop(1, nvec, 1, unroll=8)
                def _vec(kk):
                    c16 = ci_v[pl.ds(cbase + kk * 16, 16)]
                    emit(j * M + kk * 16, kk * 16, c16)

            pltpu.sync_copy(out_v, out_hbm.at[pl.ds(xoff, _G * M)])
            return carry

        lax.fori_loop(0, chunks, chunk_body, 0)

    return sc_kernel


def kernel(x, other_inputs_0, theta0, theta1, choice_indices):
    B, S, M = x.shape
    R = B * S
    sc = _make_sc_kernel(R, M)
    out = sc(
        x.reshape(-1),
        other_inputs_0.reshape(-1),
        theta0,
        theta1,
        choice_indices.reshape(-1),
    )
    return out.reshape(B, S, M)


# speedup vs baseline: 1.4842x; 1.4842x over previous
import functools

import jax
import jax.numpy as jnp
from jax import lax
from jax.experimental import pallas as pl
from jax.experimental.pallas import tpu as pltpu
from jax.experimental.pallas import tpu_sc as plsc

_TEMP = 10.0
_NC = 2
_NS = 16
_G = 4


def _make_sc_kernel(R, M):
    Mc = M - 2
    NW = _NC * _NS
    rows_per_w = R // NW
    chunks = rows_per_w // _G
    nvec = M // 16

    mesh = plsc.VectorSubcoreMesh(core_axis_name="c", subcore_axis_name="s")

    buf_types = [
        pltpu.VMEM((_G * M,), jnp.float32),
        pltpu.VMEM((_G * M,), jnp.float32),
        pltpu.VMEM((16 + _G * Mc,), jnp.int32),
        pltpu.VMEM((_G * M,), jnp.float32),
    ]

    @functools.partial(
        pl.kernel,
        out_type=jax.ShapeDtypeStruct((R * M,), jnp.float32),
        mesh=mesh,
        scratch_types=buf_types + buf_types + [
            pltpu.VMEM((M,), jnp.float32),
            pltpu.VMEM((M,), jnp.float32),
            pltpu.SemaphoreType.DMA,
            pltpu.SemaphoreType.DMA,
            pltpu.SemaphoreType.DMA,
            pltpu.SemaphoreType.DMA,
        ],
    )
    def sc_kernel(x_hbm, o_hbm, t0_hbm, t1_hbm, ci_hbm, out_hbm,
                  x_a, o_a, ci_a, out_a, x_b, o_b, ci_b, out_b,
                  t0_v, t1_v, sin_a, sin_b, sout_a, sout_b):
        bufs = ((x_a, o_a, ci_a, out_a, sin_a, sout_a),
                (x_b, o_b, ci_b, out_b, sin_b, sout_b))
        wid = lax.axis_index("s") * _NC + lax.axis_index("c")
        row_base = wid * rows_per_w
        pltpu.sync_copy(t0_hbm, t0_v)
        pltpu.sync_copy(t1_hbm, t1_v)

        @plsc.parallel_loop(0, M, 16, unroll=8)
        def _scale(i):
            t0_v[pl.ds(i, 16)] = t0_v[pl.ds(i, 16)] * _TEMP
            t1_v[pl.ds(i, 16)] = t1_v[pl.ds(i, 16)] * _TEMP

        iota = lax.iota(jnp.int32, 16)
        ci_a[pl.ds(0, 16)] = jnp.zeros((16,), jnp.int32)
        ci_b[pl.ds(0, 16)] = jnp.zeros((16,), jnp.int32)

        def in_descs(g, xbuf, obuf, cibuf, sem):
            row0 = row_base + g * _G
            return (
                (x_hbm.at[pl.ds(row0 * M, _G * M)], xbuf, sem),
                (o_hbm.at[pl.ds(row0 * M, _G * M)], obuf, sem),
                (ci_hbm.at[pl.ds(row0 * Mc, _G * Mc)],
                 cibuf.at[pl.ds(16, _G * Mc)], sem),
            )

        def start_in(g, xbuf, obuf, cibuf, sem):
            for src, dst, s in in_descs(g, xbuf, obuf, cibuf, sem):
                pltpu.async_copy(src, dst, s)

        def wait_in(g, xbuf, obuf, cibuf, sem):
            for src, dst, s in in_descs(g, xbuf, obuf, cibuf, sem):
                pltpu.make_async_copy(src, dst, s).wait()

        def out_desc(g, outbuf, sem):
            row0 = row_base + g * _G
            return (outbuf, out_hbm.at[pl.ds(row0 * M, _G * M)], sem)

        def compute(xbuf, obuf, cibuf, outbuf):
            def emit(off, moff, c16):
                xv = xbuf[pl.ds(off, 16)]
                ov = obuf[pl.ds(off, 16)]
                t0 = t0_v[pl.ds(moff, 16)]
                t1 = t1_v[pl.ds(moff, 16)]
                cz = c16 == 0
                v = jnp.where(cz, xv, ov)
                th = jnp.where(cz, t0, t1)
                denom = 1.0 + jnp.exp(th - _TEMP * jnp.abs(v))
                outbuf[pl.ds(off, 16)] = v / denom

            for j in range(_G):
                cbase = 14 + j * Mc
                c0 = cibuf[pl.ds(cbase, 16)]
                c0 = jnp.where(iota >= 2, c0, 0)
                emit(j * M, 0, c0)

                @plsc.parallel_loop(1, nvec, 1, unroll=8)
                def _vec(kk):
                    c16 = cibuf[pl.ds(cbase + kk * 16, 16)]
                    emit(j * M + kk * 16, kk * 16, c16)

        start_in(0, x_a, o_a, ci_a, sin_a)

        def body(g2, carry):
            for side in range(2):
                g = 2 * g2 + side
                xbuf, obuf, cibuf, outbuf, sin, sout = bufs[side]
                nxbuf, nobuf, ncibuf, _, nsin, _ = bufs[1 - side]
                wait_in(g, xbuf, obuf, cibuf, sin)

                @pl.when(g + 1 < chunks)
                def _():
                    start_in(g + 1, nxbuf, nobuf, ncibuf, nsin)

                @pl.when(g2 > 0)
                def _():
                    src, dst, s = out_desc(g - 2, outbuf, sout)
                    pltpu.make_async_copy(src, dst, s).wait()

                compute(xbuf, obuf, cibuf, outbuf)
                src, dst, s = out_desc(g, outbuf, sout)
                pltpu.async_copy(src, dst, s)
            return carry

        lax.fori_loop(0, chunks // 2, body, 0)
        pltpu.make_async_copy(*out_desc(chunks - 2, out_a, sout_a)).wait()
        pltpu.make_async_copy(*out_desc(chunks - 1, out_b, sout_b)).wait()

    return sc_kernel


def kernel(x, other_inputs_0, theta0, theta1, choice_indices):
    B, S, M = x.shape
    R = B * S
    sc = _make_sc_kernel(R, M)
    out = sc(
        x.reshape(-1),
        other_inputs_0.reshape(-1),
        theta0,
        theta1,
        choice_indices.reshape(-1),
    )
    return out.reshape(B, S, M)


# speedup vs baseline: 1.5950x; 1.0747x over previous
import functools

import jax
import jax.numpy as jnp
from jax import lax
from jax.experimental import pallas as pl
from jax.experimental.pallas import tpu as pltpu
from jax.experimental.pallas import tpu_sc as plsc

_TEMP = 10.0
_NC = 2
_NS = 16
_G = 4


def _make_sc_kernel(R, M):
    Mc = M - 2
    NW = _NC * _NS
    rows_per_w = R // NW
    chunks = rows_per_w // _G
    nvec = M // 16

    mesh = plsc.VectorSubcoreMesh(core_axis_name="c", subcore_axis_name="s")

    buf_types = [
        pltpu.VMEM((_G * M,), jnp.float32),
        pltpu.VMEM((_G * M,), jnp.float32),
        pltpu.VMEM((16 + _G * Mc,), jnp.int32),
        pltpu.VMEM((_G * M,), jnp.float32),
    ]

    @functools.partial(
        pl.kernel,
        out_type=jax.ShapeDtypeStruct((R * M,), jnp.float32),
        mesh=mesh,
        scratch_types=buf_types + buf_types + [
            pltpu.VMEM((M,), jnp.float32),
            pltpu.VMEM((M,), jnp.float32),
            pltpu.SemaphoreType.DMA,
            pltpu.SemaphoreType.DMA,
            pltpu.SemaphoreType.DMA,
            pltpu.SemaphoreType.DMA,
        ],
    )
    def sc_kernel(x_hbm, o_hbm, t0_hbm, t1_hbm, ci_hbm, out_hbm,
                  x_a, o_a, ci_a, out_a, x_b, o_b, ci_b, out_b,
                  t0_v, t1_v, sin_a, sin_b, sout_a, sout_b):
        bufs = ((x_a, o_a, ci_a, out_a, sin_a, sout_a),
                (x_b, o_b, ci_b, out_b, sin_b, sout_b))
        wid = lax.axis_index("s") * _NC + lax.axis_index("c")
        row_base = wid * rows_per_w
        pltpu.sync_copy(t0_hbm, t0_v)
        pltpu.sync_copy(t1_hbm, t1_v)

        @plsc.parallel_loop(0, M, 16, unroll=8)
        def _scale(i):
            t0_v[pl.ds(i, 16)] = t0_v[pl.ds(i, 16)] * _TEMP
            t1_v[pl.ds(i, 16)] = t1_v[pl.ds(i, 16)] * _TEMP

        iota = lax.iota(jnp.int32, 16)
        ci_a[pl.ds(0, 16)] = jnp.zeros((16,), jnp.int32)
        ci_b[pl.ds(0, 16)] = jnp.zeros((16,), jnp.int32)

        def in_descs(g, xbuf, obuf, cibuf, sem):
            row0 = row_base + g * _G
            return (
                (x_hbm.at[pl.ds(row0 * M, _G * M)], xbuf, sem),
                (o_hbm.at[pl.ds(row0 * M, _G * M)], obuf, sem),
                (ci_hbm.at[pl.ds(row0 * Mc, _G * Mc)],
                 cibuf.at[pl.ds(16, _G * Mc)], sem),
            )

        def start_in(g, xbuf, obuf, cibuf, sem):
            for src, dst, s in in_descs(g, xbuf, obuf, cibuf, sem):
                pltpu.async_copy(src, dst, s)

        def wait_in(g, xbuf, obuf, cibuf, sem):
            for src, dst, s in in_descs(g, xbuf, obuf, cibuf, sem):
                pltpu.make_async_copy(src, dst, s).wait()

        def out_desc(g, outbuf, sem):
            row0 = row_base + g * _G
            return (outbuf, out_hbm.at[pl.ds(row0 * M, _G * M)], sem)

        def compute(xbuf, obuf, cibuf, outbuf):
            def emit(off, c16, t0, t1):
                xv = xbuf[pl.ds(off, 16)]
                ov = obuf[pl.ds(off, 16)]
                cz = c16 == 0
                v = jnp.where(cz, xv, ov)
                th = jnp.where(cz, t0, t1)
                denom = 1.0 + jnp.exp(th - _TEMP * jnp.abs(v))
                outbuf[pl.ds(off, 16)] = v / denom

            t0_head = t0_v[pl.ds(0, 16)]
            t1_head = t1_v[pl.ds(0, 16)]
            for j in range(_G):
                c0 = cibuf[pl.ds(14 + j * Mc, 16)]
                c0 = jnp.where(iota >= 2, c0, 0)
                emit(j * M, c0, t0_head, t1_head)

            @plsc.parallel_loop(1, nvec, 1, unroll=2)
            def _vec(kk):
                moff = kk * 16
                t0 = t0_v[pl.ds(moff, 16)]
                t1 = t1_v[pl.ds(moff, 16)]
                for j in range(_G):
                    c16 = cibuf[pl.ds(14 + j * Mc + moff, 16)]
                    emit(j * M + moff, c16, t0, t1)

        start_in(0, x_a, o_a, ci_a, sin_a)

        def body(g2, carry):
            for side in range(2):
                g = 2 * g2 + side
                xbuf, obuf, cibuf, outbuf, sin, sout = bufs[side]
                nxbuf, nobuf, ncibuf, _, nsin, _ = bufs[1 - side]
                wait_in(g, xbuf, obuf, cibuf, sin)

                @pl.when(g + 1 < chunks)
                def _():
                    start_in(g + 1, nxbuf, nobuf, ncibuf, nsin)

                @pl.when(g2 > 0)
                def _():
                    src, dst, s = out_desc(g - 2, outbuf, sout)
                    pltpu.make_async_copy(src, dst, s).wait()

                compute(xbuf, obuf, cibuf, outbuf)
                src, dst, s = out_desc(g, outbuf, sout)
                pltpu.async_copy(src, dst, s)
            return carry

        lax.fori_loop(0, chunks // 2, body, 0)
        pltpu.make_async_copy(*out_desc(chunks - 2, out_a, sout_a)).wait()
        pltpu.make_async_copy(*out_desc(chunks - 1, out_b, sout_b)).wait()

    return sc_kernel


def kernel(x, other_inputs_0, theta0, theta1, choice_indices):
    B, S, M = x.shape
    R = B * S
    sc = _make_sc_kernel(R, M)
    out = sc(
        x.reshape(-1),
        other_inputs_0.reshape(-1),
        theta0,
        theta1,
        choice_indices.reshape(-1),
    )
    return out.reshape(B, S, M)


# speedup vs baseline: 2.7667x; 1.7346x over previous
import functools

import jax
import jax.numpy as jnp
from jax import lax
from jax.experimental import pallas as pl
from jax.experimental.pallas import tpu as pltpu
from jax.experimental.pallas import tpu_sc as plsc

_TEMP = 10.0
_NC = 2
_NS = 16
_CR = 8
_CC = 1024


def _make_sc_kernel(R, M):
    NW = _NC * _NS
    rows_per_w = R // NW
    chunks = (rows_per_w // _CR) * (M // _CC)
    col_halves = M // _CC
    nvec = _CC // 16

    mesh = plsc.VectorSubcoreMesh(core_axis_name="c", subcore_axis_name="s")

    buf_types = [
        pltpu.VMEM((_CR, _CC), jnp.float32),
        pltpu.VMEM((_CR, _CC), jnp.float32),
        pltpu.VMEM((_CR, _CC), jnp.int32),
        pltpu.VMEM((_CR, _CC), jnp.float32),
    ]

    @functools.partial(
        pl.kernel,
        out_type=jax.ShapeDtypeStruct((R, M), jnp.float32),
        mesh=mesh,
        compiler_params=pltpu.CompilerParams(use_tc_tiling_on_sc=True),
        scratch_types=buf_types + buf_types + [
            pltpu.VMEM((M,), jnp.float32),
            pltpu.VMEM((M,), jnp.float32),
            pltpu.SemaphoreType.DMA,
            pltpu.SemaphoreType.DMA,
            pltpu.SemaphoreType.DMA,
            pltpu.SemaphoreType.DMA,
        ],
    )
    def sc_kernel(x_hbm, o_hbm, t0_hbm, t1_hbm, ci_hbm, out_hbm,
                  x_a, o_a, ci_a, out_a, x_b, o_b, ci_b, out_b,
                  t0_v, t1_v, sin_a, sin_b, sout_a, sout_b):
        bufs = ((x_a, o_a, ci_a, out_a, sin_a, sout_a),
                (x_b, o_b, ci_b, out_b, sin_b, sout_b))
        wid = lax.axis_index("s") * _NC + lax.axis_index("c")
        row_base = wid * rows_per_w
        pltpu.sync_copy(t0_hbm, t0_v)
        pltpu.sync_copy(t1_hbm, t1_v)

        @plsc.parallel_loop(0, M, 16, unroll=8)
        def _scale(i):
            t0_v[pl.ds(i, 16)] = t0_v[pl.ds(i, 16)] * _TEMP
            t1_v[pl.ds(i, 16)] = t1_v[pl.ds(i, 16)] * _TEMP

        def chunk_origin(g):
            row0 = row_base + (g // col_halves) * _CR
            cb = (g % col_halves) * _CC
            return row0, cb

        def in_descs(g, xbuf, obuf, cibuf, sem):
            row0, cb = chunk_origin(g)
            return (
                (x_hbm.at[pl.ds(row0, _CR), pl.ds(cb, _CC)], xbuf, sem),
                (o_hbm.at[pl.ds(row0, _CR), pl.ds(cb, _CC)], obuf, sem),
                (ci_hbm.at[pl.ds(row0, _CR), pl.ds(cb, _CC)], cibuf, sem),
            )

        def start_in(g, xbuf, obuf, cibuf, sem):
            for src, dst, s in in_descs(g, xbuf, obuf, cibuf, sem):
                pltpu.async_copy(src, dst, s)

        def wait_in(g, xbuf, obuf, cibuf, sem):
            for src, dst, s in in_descs(g, xbuf, obuf, cibuf, sem):
                pltpu.make_async_copy(src, dst, s).wait()

        def out_desc(g, outbuf, sem):
            row0, cb = chunk_origin(g)
            return (outbuf, out_hbm.at[pl.ds(row0, _CR), pl.ds(cb, _CC)], sem)

        def compute(g, xbuf, obuf, cibuf, outbuf):
            _, cb = chunk_origin(g)

            @plsc.parallel_loop(0, nvec, 1, unroll=2)
            def _vec(kk):
                moff = kk * 16
                t0 = t0_v[pl.ds(cb + moff, 16)]
                t1 = t1_v[pl.ds(cb + moff, 16)]
                for s in range(_CR):
                    c16 = cibuf[s, pl.ds(moff, 16)]
                    xv = xbuf[s, pl.ds(moff, 16)]
                    ov = obuf[s, pl.ds(moff, 16)]
                    cz = c16 == 0
                    v = jnp.where(cz, xv, ov)
                    th = jnp.where(cz, t0, t1)
                    denom = 1.0 + jnp.exp(th - _TEMP * jnp.abs(v))
                    outbuf[s, pl.ds(moff, 16)] = v / denom

        start_in(0, x_a, o_a, ci_a, sin_a)

        def body(g2, carry):
            for side in range(2):
                g = 2 * g2 + side
                xbuf, obuf, cibuf, outbuf, sin, sout = bufs[side]
                nxbuf, nobuf, ncibuf, _, nsin, _ = bufs[1 - side]
                wait_in(g, xbuf, obuf, cibuf, sin)

                @pl.when(g + 1 < chunks)
                def _():
                    start_in(g + 1, nxbuf, nobuf, ncibuf, nsin)

                @pl.when(g2 > 0)
                def _():
                    src, dst, s = out_desc(g - 2, outbuf, sout)
                    pltpu.make_async_copy(src, dst, s).wait()

                compute(g, xbuf, obuf, cibuf, outbuf)
                src, dst, s = out_desc(g, outbuf, sout)
                pltpu.async_copy(src, dst, s)
            return carry

        lax.fori_loop(0, chunks // 2, body, 0)
        pltpu.make_async_copy(*out_desc(chunks - 2, out_a, sout_a)).wait()
        pltpu.make_async_copy(*out_desc(chunks - 1, out_b, sout_b)).wait()

    return sc_kernel


def kernel(x, other_inputs_0, theta0, theta1, choice_indices):
    B, S, M = x.shape
    R = B * S
    cip = jnp.pad(choice_indices, ((0, 0), (0, 0), (2, 0)))
    sc = _make_sc_kernel(R, M)
    out = sc(
        x.reshape(R, M),
        other_inputs_0.reshape(R, M),
        theta0,
        theta1,
        cip.reshape(R, M),
    )
    return out.reshape(B, S, M)
